# baseline (speedup 1.0000x reference)
import math

import jax
import jax.numpy as jnp
from jax.experimental import pallas as pl

E = 16
HID = 24
TOPK = 2
BLK = 2048


def _moe_body(x_ref, wr_ref, br_ref, w1_ref, b1_ref, w2_ref, b2_ref, o_ref):
    xb = x_ref[...]
    logits = jnp.dot(xb, wr_ref[...], preferred_element_type=jnp.float32)
    logits = logits + br_ref[...]
    mx = jnp.max(logits, axis=-1, keepdims=True)
    p = jnp.exp(logits - mx)
    p = p / jnp.sum(p, axis=-1, keepdims=True)

    eids = jax.lax.broadcasted_iota(jnp.int32, p.shape, 1)
    m1 = jnp.max(p, axis=-1, keepdims=True)
    i1 = jnp.min(jnp.where(p >= m1, eids, E), axis=-1, keepdims=True)
    pm = jnp.where(eids == i1, -1.0, p)
    m2 = jnp.max(pm, axis=-1, keepdims=True)
    i2 = jnp.min(jnp.where(pm >= m2, eids, E), axis=-1, keepdims=True)
    s = m1 + m2 + 1e-8
    w_e = jnp.where(eids == i1, m1, 0.0) + jnp.where(eids == i2, m2, 0.0)
    w_e = w_e / s

    rr = jax.lax.broadcasted_iota(jnp.int32, (E, E * HID), 0)
    rc = jax.lax.broadcasted_iota(jnp.int32, (E, E * HID), 1) // HID
    expand = (rr == rc).astype(jnp.float32)
    wfull = jnp.dot(w_e, expand, preferred_element_type=jnp.float32)

    h = jnp.dot(xb.astype(jnp.bfloat16), w1_ref[...],
                preferred_element_type=jnp.float32)
    h = h + b1_ref[...]
    h = 0.5 * h * (1.0 + jax.lax.erf(h * (1.0 / math.sqrt(2.0))))
    h = h * wfull
    out = jnp.dot(h.astype(jnp.bfloat16), w2_ref[...],
                  preferred_element_type=jnp.float32)
    out = out + jnp.dot(w_e, b2_ref[...], preferred_element_type=jnp.float32)
    o_ref[...] = out


def kernel(x, Wr, br, W1, b1, W2, b2):
    Bn, Hn, Wn, Cn = x.shape
    N = Bn * Hn * Wn
    xt = x.reshape(N, Cn)
    W1s = W1.transpose(2, 0, 1).reshape(Cn, E * HID).astype(jnp.bfloat16)
    b1s = b1.reshape(1, E * HID)
    W2s = W2.transpose(0, 2, 1).reshape(E * HID, Cn).astype(jnp.bfloat16)
    WrT = Wr.T
    brr = br.reshape(1, E)

    out = pl.pallas_call(
        _moe_body,
        grid=(N // BLK,),
        in_specs=[
            pl.BlockSpec((BLK, Cn), lambda i: (i, 0)),
            pl.BlockSpec((Cn, E), lambda i: (0, 0)),
            pl.BlockSpec((1, E), lambda i: (0, 0)),
            pl.BlockSpec((Cn, E * HID), lambda i: (0, 0)),
            pl.BlockSpec((1, E * HID), lambda i: (0, 0)),
            pl.BlockSpec((E * HID, Cn), lambda i: (0, 0)),
            pl.BlockSpec((E, Cn), lambda i: (0, 0)),
        ],
        out_specs=pl.BlockSpec((BLK, Cn), lambda i: (i, 0)),
        out_shape=jax.ShapeDtypeStruct((N, Cn), x.dtype),
    )(xt, WrT, brr, W1s, b1s, W2s, b2)
    return out.reshape(Bn, Hn, Wn, Cn)


# speedup vs baseline: 1.5246x; 1.5246x over previous
import math

import jax
import jax.numpy as jnp
from jax.experimental import pallas as pl

E = 16
HID = 24
TOPK = 2
BLK = 2048


def _moe_body(x_ref, wr_ref, br_ref, w1_ref, b1_ref, w2_ref, b2_ref, o_ref):
    xb = x_ref[...]
    logits = jnp.dot(xb, wr_ref[...], preferred_element_type=jnp.float32)
    logits = logits + br_ref[...]
    l1 = jnp.max(logits, axis=-1, keepdims=True)
    lm = jnp.where(logits >= l1, -jnp.inf, logits)
    l2 = jnp.max(lm, axis=-1, keepdims=True)
    mask = (logits >= l2).astype(jnp.float32)
    ex = jnp.exp(logits - l1) * mask
    w_e = ex / jnp.sum(ex, axis=-1, keepdims=True)

    rr = jax.lax.broadcasted_iota(jnp.int32, (E, E * HID), 0)
    rc = jax.lax.broadcasted_iota(jnp.int32, (E, E * HID), 1) // HID
    expand = jnp.where(rr == rc, 0.5, 0.0)
    wfull = jnp.dot(w_e, expand, preferred_element_type=jnp.float32)

    h = jnp.dot(xb.astype(jnp.bfloat16), w1_ref[...],
                preferred_element_type=jnp.float32)
    h = h + b1_ref[...]
    u = h * wfull
    t = jax.lax.erf(h * (1.0 / math.sqrt(2.0)))
    g = u + u * t
    out = jnp.dot(g.astype(jnp.bfloat16), w2_ref[...],
                  preferred_element_type=jnp.float32)
    out = out + jnp.dot(w_e, b2_ref[...], preferred_element_type=jnp.float32)
    o_ref[...] = out


def kernel(x, Wr, br, W1, b1, W2, b2):
    Bn, Hn, Wn, Cn = x.shape
    N = Bn * Hn * Wn
    xt = x.reshape(N, Cn)
    W1s = W1.transpose(2, 0, 1).reshape(Cn, E * HID).astype(jnp.bfloat16)
    b1s = b1.reshape(1, E * HID)
    W2s = W2.transpose(0, 2, 1).reshape(E * HID, Cn).astype(jnp.bfloat16)
    WrT = Wr.T
    brr = br.reshape(1, E)

    out = pl.pallas_call(
        _moe_body,
        grid=(N // BLK,),
        in_specs=[
            pl.BlockSpec((BLK, Cn), lambda i: (i, 0)),
            pl.BlockSpec((Cn, E), lambda i: (0, 0)),
            pl.BlockSpec((1, E), lambda i: (0, 0)),
            pl.BlockSpec((Cn, E * HID), lambda i: (0, 0)),
            pl.BlockSpec((1, E * HID), lambda i: (0, 0)),
            pl.BlockSpec((E * HID, Cn), lambda i: (0, 0)),
            pl.BlockSpec((E, Cn), lambda i: (0, 0)),
        ],
        out_specs=pl.BlockSpec((BLK, Cn), lambda i: (i, 0)),
        out_shape=jax.ShapeDtypeStruct((N, Cn), x.dtype),
    )(xt, WrT, brr, W1s, b1s, W2s, b2)
    return out.reshape(Bn, Hn, Wn, Cn)
